# Initial kernel scaffold; baseline (speedup 1.0000x reference)
import jax
import jax.numpy as jnp
from jax.experimental import pallas as pl


def kernel(input, table):
    raise NotImplementedError("write your pallas kernel here")


# speedup vs baseline: 2.2779x; 2.2779x over previous
import functools

import jax
import jax.numpy as jnp
from jax import lax
from jax.experimental import pallas as pl
from jax.experimental.pallas import tpu as pltpu
from jax.experimental.pallas import tpu_sc as plsc

_PAD = 0
_NUM_EMB = 256
_D = 128
_B = 4096
_S = 200

_NC = 2
_NS = 16
_NW = _NC * _NS
_CH = 100
_NROWS = (_B * _S) // _CH
_RPW = _NROWS // _NW
_BB = 512


def _pos_body(inp_ref, tab_ref, pos_ref, tabz_ref):
    x = inp_ref[...]
    mask = (x != _PAD).astype(jnp.int32)
    m_bf = mask.astype(jnp.bfloat16)
    row = lax.broadcasted_iota(jnp.int32, (_S, _S), 0)
    col = lax.broadcasted_iota(jnp.int32, (_S, _S), 1)
    tri = (row <= col).astype(jnp.bfloat16)
    csum = lax.dot_general(m_bf, tri, (((1,), (0,)), ((), ())),
                           preferred_element_type=jnp.float32)
    pos_ref[...] = csum.astype(jnp.int32) * mask
    r = lax.broadcasted_iota(jnp.int32, (_NUM_EMB, _D), 0)
    tabz_ref[...] = jnp.where(r == _PAD, 0.0, tab_ref[...])


_pos_call = pl.pallas_call(
    _pos_body,
    grid=(_B // _BB,),
    in_specs=[
        pl.BlockSpec((_BB, _S), lambda i: (i, 0)),
        pl.BlockSpec((_NUM_EMB, _D), lambda i: (0, 0)),
    ],
    out_specs=[
        pl.BlockSpec((_BB, _S), lambda i: (i, 0)),
        pl.BlockSpec((_NUM_EMB, _D), lambda i: (0, 0)),
    ],
    out_shape=[
        jax.ShapeDtypeStruct((_B, _S), jnp.int32),
        jax.ShapeDtypeStruct((_NUM_EMB, _D), jnp.float32),
    ],
)


def _gather_body(pos_hbm, tab_hbm, out_hbm, idx_v, row_v, sem):
    wid = lax.axis_index("s") * _NC + lax.axis_index("c")
    base = wid * _RPW
    pltpu.sync_copy(pos_hbm.at[pl.ds(base, _RPW)], idx_v)

    def step(i, carry):
        pltpu.async_copy(tab_hbm.at[idx_v.at[i]], row_v, sem).wait()
        pltpu.sync_copy(row_v, out_hbm.at[base + i])
        return carry

    lax.fori_loop(0, _RPW, step, 0)


@functools.lru_cache(maxsize=None)
def _make_gather_call():
    return functools.partial(
        pl.kernel,
        out_type=jax.ShapeDtypeStruct((_NROWS, _CH, _D), jnp.float32),
        mesh=plsc.VectorSubcoreMesh(core_axis_name="c", subcore_axis_name="s"),
        scratch_types=[
            pltpu.VMEM((_RPW, _CH), jnp.int32),
            pltpu.VMEM((_CH, _D), jnp.float32),
            pltpu.SemaphoreType.DMA,
        ],
    )(_gather_body)


def kernel(input, table):
    _gather_call = _make_gather_call()
    pos, tabz = _pos_call(input, table)
    pos2 = pos.reshape(_NROWS, _CH)
    out = _gather_call(pos2, tabz)
    return out.reshape(_B, _S, _D)


# speedup vs baseline: 2.2965x; 1.0082x over previous
import functools

import jax
import jax.numpy as jnp
from jax import lax
from jax.experimental import pallas as pl
from jax.experimental.pallas import tpu as pltpu
from jax.experimental.pallas import tpu_sc as plsc

_PAD = 0
_NUM_EMB = 256
_D = 128
_B = 4096
_S = 200

_NC = 2
_NS = 16
_NW = _NC * _NS
_CH = 100
_NROWS = (_B * _S) // _CH
_RPW = _NROWS // _NW
_BB = 512


def _pos_body(inp_ref, tab_ref, pos_ref, tabz_ref):
    x = inp_ref[...]
    mask = (x != _PAD).astype(jnp.int32)
    m_bf = mask.astype(jnp.bfloat16)
    row = lax.broadcasted_iota(jnp.int32, (_S, _S), 0)
    col = lax.broadcasted_iota(jnp.int32, (_S, _S), 1)
    tri = (row <= col).astype(jnp.bfloat16)
    csum = lax.dot_general(m_bf, tri, (((1,), (0,)), ((), ())),
                           preferred_element_type=jnp.float32)
    pos_ref[...] = csum.astype(jnp.int32) * mask
    r = lax.broadcasted_iota(jnp.int32, (_NUM_EMB, _D), 0)
    tabz_ref[...] = jnp.where(r == _PAD, 0.0, tab_ref[...])


_pos_call = pl.pallas_call(
    _pos_body,
    grid=(_B // _BB,),
    in_specs=[
        pl.BlockSpec((_BB, _S), lambda i: (i, 0)),
        pl.BlockSpec((_NUM_EMB, _D), lambda i: (0, 0)),
    ],
    out_specs=[
        pl.BlockSpec((_BB, _S), lambda i: (i, 0)),
        pl.BlockSpec((_NUM_EMB, _D), lambda i: (0, 0)),
    ],
    out_shape=[
        jax.ShapeDtypeStruct((_B, _S), jnp.int32),
        jax.ShapeDtypeStruct((_NUM_EMB, _D), jnp.float32),
    ],
)


_NBUF = 4
_NITER = _RPW // _NBUF


def _gather_body(pos_hbm, tab_hbm, out_hbm, idx_v, bufs, sems):
    wid = lax.axis_index("s") * _NC + lax.axis_index("c")
    base = wid * _RPW
    pltpu.sync_copy(pos_hbm.at[pl.ds(base, _RPW)], idx_v)

    def gather(i, b):
        pltpu.async_copy(tab_hbm.at[idx_v.at[i]], bufs.at[b], sems.at[b])

    def write(i, b):
        pltpu.async_copy(bufs.at[b], out_hbm.at[base + i], sems.at[b])

    def wait(b):
        pltpu.make_async_copy(out_hbm.at[base], bufs.at[b], sems.at[b]).wait()

    for b in range(2):
        gather(b, b)

    def step(k, carry):
        r0 = k * _NBUF
        for b in range(2):
            wait(b)
        for b in range(2):
            write(r0 + b, b)

        @pl.when(k > 0)
        def _():
            for b in range(2, 4):
                wait(b)

        for b in range(2, 4):
            gather(r0 + b, b)
        for b in range(2, 4):
            wait(b)
        for b in range(2, 4):
            write(r0 + b, b)
        for b in range(2):
            wait(b)

        @pl.when(k < _NITER - 1)
        def _():
            for b in range(2):
                gather(r0 + _NBUF + b, b)

        return carry

    lax.fori_loop(0, _NITER, step, 0)
    for b in range(2, 4):
        wait(b)


@functools.lru_cache(maxsize=None)
def _make_gather_call():
    return functools.partial(
        pl.kernel,
        out_type=jax.ShapeDtypeStruct((_NROWS, _CH, _D), jnp.float32),
        mesh=plsc.VectorSubcoreMesh(core_axis_name="c", subcore_axis_name="s"),
        scratch_types=[
            pltpu.VMEM((_RPW, _CH), jnp.int32),
            pltpu.VMEM((_NBUF, _CH, _D), jnp.float32),
            pltpu.SemaphoreType.DMA((_NBUF,)),
        ],
    )(_gather_body)


def kernel(input, table):
    _gather_call = _make_gather_call()
    pos, tabz = _pos_call(input, table)
    pos2 = pos.reshape(_NROWS, _CH)
    out = _gather_call(pos2, tabz)
    return out.reshape(_B, _S, _D)


# speedup vs baseline: 10.6071x; 4.6188x over previous
import functools

import jax
import jax.numpy as jnp
from jax import lax
from jax.experimental import pallas as pl
from jax.experimental.pallas import tpu as pltpu
from jax.experimental.pallas import tpu_sc as plsc

_PAD = 0
_NUM_EMB = 256
_D = 128
_B = 4096
_S = 200

_NC = 2
_NS = 16
_NW = _NC * _NS
_RPW = _B // _NW
_BB = 512


def _pos_body(inp_ref, tab_ref, pos_ref, tabz_ref):
    x = inp_ref[...]
    mask = (x != _PAD).astype(jnp.int32)
    m_bf = mask.astype(jnp.bfloat16)
    row = lax.broadcasted_iota(jnp.int32, (_S, _S), 0)
    col = lax.broadcasted_iota(jnp.int32, (_S, _S), 1)
    tri = (row <= col).astype(jnp.bfloat16)
    csum = lax.dot_general(m_bf, tri, (((1,), (0,)), ((), ())),
                           preferred_element_type=jnp.float32)
    pos_ref[...] = csum.astype(jnp.int32) * mask
    r = lax.broadcasted_iota(jnp.int32, (_NUM_EMB, _D), 0)
    tabz_ref[...] = jnp.where(r == _PAD, 0.0, tab_ref[...])


_pos_call = pl.pallas_call(
    _pos_body,
    grid=(_B // _BB,),
    in_specs=[
        pl.BlockSpec((_BB, _S), lambda i: (i, 0)),
        pl.BlockSpec((_NUM_EMB, _D), lambda i: (0, 0)),
    ],
    out_specs=[
        pl.BlockSpec((_BB, _S), lambda i: (i, 0)),
        pl.BlockSpec((_NUM_EMB, _D), lambda i: (0, 0)),
    ],
    out_shape=[
        jax.ShapeDtypeStruct((_B, _S), jnp.int32),
        jax.ShapeDtypeStruct((_NUM_EMB, _D), jnp.float32),
    ],
)


def _gather_body(pos_hbm, tab_hbm, out_hbm, idx_v, tab_v, row_v, fsem, wsem):
    wid = lax.axis_index("s") * _NC + lax.axis_index("c")
    base = wid * _RPW
    n = _RPW * _S
    pltpu.sync_copy(pos_hbm.at[pl.ds(base * _S, n)], idx_v.at[pl.ds(0, n)])
    pltpu.sync_copy(tab_hbm, tab_v)
    slab = tab_v.at[pl.ds(1, _S)]

    def copy_elem(pv, l, j):
        p = pv[l]
        for k in range(_D // 16):
            row_v[j, pl.ds(k * 16, 16)] = tab_v[p, pl.ds(k * 16, 16)]

    def rebuild(r, gb):
        rb = r * _S

        def chunk(c, carry):
            pv = idx_v[pl.ds(rb + c * 16, 16)]
            for l in range(16):
                copy_elem(pv, l, c * 16 + l)
            return carry

        lax.fori_loop(0, _S // 16, chunk, 0)
        ptail = idx_v[pl.ds(rb + (_S // 16) * 16, 16)]
        for l in range(_S - (_S // 16) * 16):
            copy_elem(ptail, l, (_S // 16) * 16 + l)
        pltpu.async_copy(row_v, out_hbm.at[gb], wsem).wait()

    def step(r, nfast):
        gb = base + r
        tail = idx_v[pl.ds(r * _S + _S - 16, 16)]
        fast = tail[15] == _S

        @pl.when(fast)
        def _():
            pltpu.async_copy(slab, out_hbm.at[gb], fsem)

        @pl.when(jnp.logical_not(fast))
        def _():
            rebuild(r, gb)

        return nfast + fast.astype(jnp.int32)

    nfast = lax.fori_loop(0, _RPW, step, 0)

    def drain(i, c):
        pltpu.make_async_copy(slab, out_hbm.at[base], fsem).wait()
        return c

    lax.fori_loop(0, nfast, drain, 0)


@functools.lru_cache(maxsize=None)
def _make_gather_call():
    return functools.partial(
        pl.kernel,
        out_type=jax.ShapeDtypeStruct((_B, _S, _D), jnp.float32),
        mesh=plsc.VectorSubcoreMesh(core_axis_name="c", subcore_axis_name="s"),
        scratch_types=[
            pltpu.VMEM((_RPW * _S + 16,), jnp.int32),
            pltpu.VMEM((_NUM_EMB, _D), jnp.float32),
            pltpu.VMEM((_S, _D), jnp.float32),
            pltpu.SemaphoreType.DMA,
            pltpu.SemaphoreType.DMA,
        ],
    )(_gather_body)


def kernel(input, table):
    _gather_call = _make_gather_call()
    pos, tabz = _pos_call(input, table)
    return _gather_call(pos.reshape(-1), tabz)


# speedup vs baseline: 10.9752x; 1.0347x over previous
import functools

import jax
import jax.numpy as jnp
from jax import lax
from jax.experimental import pallas as pl
from jax.experimental.pallas import tpu as pltpu
from jax.experimental.pallas import tpu_sc as plsc

_PAD = 0
_NUM_EMB = 256
_D = 128
_B = 4096
_S = 200

_NC = 2
_NS = 16
_NW = _NC * _NS
_RPW = _B // _NW
_NCH = _S // 16
_TOFF = _S - 16


def _body(inp_hbm, tab_hbm, out_hbm, tok_v, tab_v, row_v, fsem, wsem):
    wid = lax.axis_index("s") * _NC + lax.axis_index("c")
    base = wid * _RPW
    pltpu.sync_copy(inp_hbm.at[pl.ds(base, _RPW)], tok_v)
    pltpu.sync_copy(tab_hbm, tab_v)
    zeros = jnp.zeros((16,), jnp.float32)
    for k in range(_D // 16):
        tab_v[_PAD, pl.ds(k * 16, 16)] = zeros
    slab = tab_v.at[pl.ds(1, _S)]

    def copy_row(p, j):
        for k in range(_D // 16):
            row_v[j, pl.ds(k * 16, 16)] = tab_v[p, pl.ds(k * 16, 16)]

    def rebuild(r, gb):
        def chunk(c, carry):
            v = tok_v[r, pl.ds(c * 16, 16)]
            for l in range(16):
                t = (v[l] != _PAD).astype(jnp.int32)
                carry = carry + t
                copy_row(carry * t, c * 16 + l)
            return carry

        carry = lax.fori_loop(0, _NCH, chunk, 0)
        vt = tok_v[r, pl.ds(_TOFF, 16)]
        for l in range(16 - (_S - _NCH * 16), 16):
            t = (vt[l] != _PAD).astype(jnp.int32)
            carry = carry + t
            copy_row(carry * t, _TOFF + l)
        pltpu.async_copy(row_v, out_hbm.at[gb], wsem).wait()

    def step(r, nfast):
        gb = base + r
        acc = tok_v[r, pl.ds(0, 16)]
        for c in range(1, _NCH):
            acc = jnp.minimum(acc, tok_v[r, pl.ds(c * 16, 16)])
        acc = jnp.minimum(acc, tok_v[r, pl.ds(_TOFF, 16)])
        fast = acc[0] != _PAD
        for l in range(1, 16):
            fast = jnp.logical_and(fast, acc[l] != _PAD)

        @pl.when(fast)
        def _():
            pltpu.async_copy(slab, out_hbm.at[gb], fsem)

        @pl.when(jnp.logical_not(fast))
        def _():
            rebuild(r, gb)

        return nfast + fast.astype(jnp.int32)

    nfast = lax.fori_loop(0, _RPW, step, 0)

    def drain(i, c):
        pltpu.make_async_copy(slab, out_hbm.at[base], fsem).wait()
        return c

    lax.fori_loop(0, nfast, drain, 0)


@functools.lru_cache(maxsize=None)
def _make_call():
    return functools.partial(
        pl.kernel,
        out_type=jax.ShapeDtypeStruct((_B, _S, _D), jnp.float32),
        mesh=plsc.VectorSubcoreMesh(core_axis_name="c", subcore_axis_name="s"),
        scratch_types=[
            pltpu.VMEM((_RPW, _S), jnp.int32),
            pltpu.VMEM((_NUM_EMB, _D), jnp.float32),
            pltpu.VMEM((_S, _D), jnp.float32),
            pltpu.SemaphoreType.DMA,
            pltpu.SemaphoreType.DMA,
        ],
    )(_body)


def kernel(input, table):
    return _make_call()(input, table)


# speedup vs baseline: 12.5710x; 1.1454x over previous
import functools

import jax
import jax.numpy as jnp
from jax import lax
from jax.experimental import pallas as pl
from jax.experimental.pallas import tpu as pltpu
from jax.experimental.pallas import tpu_sc as plsc

_PAD = 0
_NUM_EMB = 256
_D = 128
_B = 4096
_S = 200

_NC = 2
_NS = 16
_NW = _NC * _NS
_RPW = _B // _NW
_NCH = _S // 16
_TOFF = _S - 16


def _body(inp_hbm, tab_hbm, out_hbm, tok_v, tab_v, row_v, fsem, wsem):
    wid = lax.axis_index("s") * _NC + lax.axis_index("c")
    base = wid * _RPW
    pltpu.sync_copy(inp_hbm.at[pl.ds(base, _RPW)], tok_v)
    pltpu.sync_copy(tab_hbm, tab_v)
    zeros = jnp.zeros((16,), jnp.float32)
    for k in range(_D // 16):
        tab_v[_PAD, pl.ds(k * 16, 16)] = zeros
    slab = tab_v.at[pl.ds(1, _S)]

    def copy_row(par, p, j):
        for k in range(_D // 16):
            row_v[par, j, pl.ds(k * 16, 16)] = tab_v[p, pl.ds(k * 16, 16)]

    def rebuild(r, gb, nslow):
        par = lax.rem(nslow, 2)

        @pl.when(nslow >= 2)
        def _():
            pltpu.make_async_copy(row_v.at[par], out_hbm.at[base], wsem).wait()

        def chunk(c, carry):
            v = tok_v[r, pl.ds(c * 16, 16)]
            for l in range(16):
                t = (v[l] != _PAD).astype(jnp.int32)
                carry = carry + t
                copy_row(par, carry * t, c * 16 + l)
            return carry

        carry = lax.fori_loop(0, _NCH, chunk, 0)
        vt = tok_v[r, pl.ds(_TOFF, 16)]
        for l in range(16 - (_S - _NCH * 16), 16):
            t = (vt[l] != _PAD).astype(jnp.int32)
            carry = carry + t
            copy_row(par, carry * t, _TOFF + l)
        pltpu.async_copy(row_v.at[par], out_hbm.at[gb], wsem)

    def step(r, counts):
        nfast, nslow = counts
        gb = base + r
        acc = tok_v[r, pl.ds(0, 16)]
        for c in range(1, _NCH):
            acc = jnp.minimum(acc, tok_v[r, pl.ds(c * 16, 16)])
        acc = jnp.minimum(acc, tok_v[r, pl.ds(_TOFF, 16)])
        fast = acc[0] != _PAD
        for l in range(1, 16):
            fast = jnp.logical_and(fast, acc[l] != _PAD)

        @pl.when(fast)
        def _():
            pltpu.async_copy(slab, out_hbm.at[gb], fsem)

        @pl.when(jnp.logical_not(fast))
        def _():
            rebuild(r, gb, nslow)

        fi = fast.astype(jnp.int32)
        return (nfast + fi, nslow + 1 - fi)

    nfast, nslow = lax.fori_loop(0, _RPW, step, (0, 0))

    def drain_f(i, c):
        pltpu.make_async_copy(slab, out_hbm.at[base], fsem).wait()
        return c

    lax.fori_loop(0, nfast, drain_f, 0)

    def drain_s(i, c):
        pltpu.make_async_copy(row_v.at[0], out_hbm.at[base], wsem).wait()
        return c

    lax.fori_loop(0, jnp.minimum(nslow, 2), drain_s, 0)


@functools.lru_cache(maxsize=None)
def _make_call():
    return functools.partial(
        pl.kernel,
        out_type=jax.ShapeDtypeStruct((_B, _S, _D), jnp.float32),
        mesh=plsc.VectorSubcoreMesh(core_axis_name="c", subcore_axis_name="s"),
        scratch_types=[
            pltpu.VMEM((_RPW, _S), jnp.int32),
            pltpu.VMEM((_NUM_EMB, _D), jnp.float32),
            pltpu.VMEM((2, _S, _D), jnp.float32),
            pltpu.SemaphoreType.DMA,
            pltpu.SemaphoreType.DMA,
        ],
    )(_body)


def kernel(input, table):
    return _make_call()(input, table)
